# speedup vs baseline: 4.0369x; 4.0369x over previous
import functools

import jax
import jax.numpy as jnp
from jax import lax
from jax.experimental import pallas as pl
from jax.experimental.pallas import tpu as pltpu
from jax.experimental.pallas import tpu_sc as plsc

N = 10000
D = 128
H = 512
G = 256

NC = 2
NS = 16
NW = NC * NS

CH = 128
N_TC = 10240
N_SP = N_TC + NS
ZROWS = N_SP // NS
OROWS = N_TC // NS

BLK = 512
NB = 256


@functools.lru_cache(maxsize=None)
def _make_sc_aggregate(EP: int):
    EPW = EP // NW
    n_chunks = EPW // CH

    mesh = plsc.VectorSubcoreMesh(
        core_axis_name="c", subcore_axis_name="s", num_cores=NC, num_subcores=NS
    )

    @functools.partial(
        pl.kernel,
        out_type=jax.ShapeDtypeStruct((NC, N_TC, D), jnp.float32),
        mesh=mesh,
        scratch_types=[
            pltpu.VMEM((CH,), jnp.int32),
            pltpu.VMEM((CH,), jnp.int32),
            pltpu.VMEM((CH, D), jnp.float32),
            pltpu.VMEM_SHARED((N_SP, D), jnp.float32),
            pltpu.SemaphoreType.DMA,
        ],
    )
    def sc_aggregate(h_hbm, src_hbm, dst_hbm, out_hbm, src_v, dst_v, rows_v, agg_sh, sem):
        c = lax.axis_index("c")
        s = lax.axis_index("s")
        wid = c * NS + s

        zero16 = jnp.zeros((16,), jnp.float32)

        def _zero_row(i, carry):
            for j in range(D // 16):
                rows_v[i, pl.ds(j * 16, 16)] = zero16
            return carry

        lax.fori_loop(0, CH, _zero_row, 0)

        zbase = s * ZROWS
        off = 0
        while off < ZROWS:
            sz = min(CH, ZROWS - off)
            pltpu.sync_copy(rows_v.at[pl.ds(0, sz), :], agg_sh.at[pl.ds(zbase + off, sz), :])
            off += sz
        plsc.subcore_barrier()

        ebase = wid * EPW

        def _body(i, carry):
            b = ebase + i * CH
            pltpu.sync_copy(src_hbm.at[pl.ds(b, CH)], src_v)
            pltpu.sync_copy(dst_hbm.at[pl.ds(b, CH)], dst_v)
            pltpu.async_copy(h_hbm.at[src_v], rows_v, sem).wait()
            pltpu.sync_copy(rows_v, agg_sh.at[dst_v], add=True)
            return carry

        lax.fori_loop(0, n_chunks, _body, 0)
        plsc.subcore_barrier()

        ob = s * OROWS
        pltpu.sync_copy(agg_sh.at[pl.ds(ob, OROWS), :], out_hbm.at[c, pl.ds(ob, OROWS), :])

    return sc_aggregate


def _mlp_body(h_ref, a0_ref, a1_ref, W1_ref, b1_ref, W2_ref, b2_ref, o_ref):
    agg = a0_ref[...] + a1_ref[...]
    z = jnp.dot(agg, W1_ref[...], preferred_element_type=jnp.float32) + b1_ref[...]
    z = jnp.maximum(z, 0.0)
    o_ref[...] = h_ref[...] + jnp.dot(z, W2_ref[...], preferred_element_type=jnp.float32) + b2_ref[...]


def _tc_mlp(h, a0, a1, W1, b1, W2, b2):
    grid = (N_TC // BLK,)
    return pl.pallas_call(
        _mlp_body,
        grid=grid,
        in_specs=[
            pl.BlockSpec((BLK, D), lambda i: (i, 0)),
            pl.BlockSpec((BLK, D), lambda i: (i, 0)),
            pl.BlockSpec((BLK, D), lambda i: (i, 0)),
            pl.BlockSpec((D, H), lambda i: (0, 0)),
            pl.BlockSpec((1, H), lambda i: (0, 0)),
            pl.BlockSpec((H, D), lambda i: (0, 0)),
            pl.BlockSpec((1, D), lambda i: (0, 0)),
        ],
        out_specs=pl.BlockSpec((BLK, D), lambda i: (i, 0)),
        out_shape=jax.ShapeDtypeStruct((N_TC, D), jnp.float32),
    )(h, a0, a1, W1, b1, W2, b2)


def _pool_body(batch_ref, h_ref, lng_ref, lnb_ref, W1_ref, b1_ref, bng_ref,
               bnb_ref, W2_ref, b2_ref, o_ref, acc_ref):
    i = pl.program_id(0)

    @pl.when(i == 0)
    def _():
        acc_ref[...] = jnp.zeros_like(acc_ref)

    b = batch_ref[...].reshape(1, NB)
    gids = lax.broadcasted_iota(jnp.int32, (G, 1), 0)
    P = (b == gids).astype(jnp.float32)
    acc_ref[...] += jnp.dot(P, h_ref[...], preferred_element_type=jnp.float32)

    @pl.when(i == pl.num_programs(0) - 1)
    def _():
        g = acc_ref[...]
        mu = jnp.mean(g, axis=1, keepdims=True)
        var = jnp.mean((g - mu) ** 2, axis=1, keepdims=True)
        z = (g - mu) * lax.rsqrt(var + 1e-5) * lng_ref[...] + lnb_ref[...]
        z = jnp.dot(z, W1_ref[...], preferred_element_type=jnp.float32) + b1_ref[...]
        z = z * (1.0 / jnp.sqrt(1.0 + 1e-5)) * bng_ref[...] + bnb_ref[...]
        z = jnp.maximum(z, 0.0)
        out = jnp.dot(z, W2_ref[...], preferred_element_type=jnp.float32) + b2_ref[...]
        o_ref[...] = jnp.clip(out, 0.0, 100.0)


def _tc_pool_head(batch3d, h, ln_g, ln_b, W_fc1, b_fc1, bn_g, bn_b, W_fc2, b_fc2):
    grid = (N_TC // NB,)
    return pl.pallas_call(
        _pool_body,
        grid=grid,
        in_specs=[
            pl.BlockSpec((1, 1, NB), lambda i: (i, 0, 0)),
            pl.BlockSpec((NB, D), lambda i: (i, 0)),
            pl.BlockSpec((1, D), lambda i: (0, 0)),
            pl.BlockSpec((1, D), lambda i: (0, 0)),
            pl.BlockSpec((D, D), lambda i: (0, 0)),
            pl.BlockSpec((1, D), lambda i: (0, 0)),
            pl.BlockSpec((1, D), lambda i: (0, 0)),
            pl.BlockSpec((1, D), lambda i: (0, 0)),
            pl.BlockSpec((D, 1), lambda i: (0, 0)),
            pl.BlockSpec((1, 1), lambda i: (0, 0)),
        ],
        out_specs=pl.BlockSpec((G, 1), lambda i: (0, 0)),
        out_shape=jax.ShapeDtypeStruct((G, 1), jnp.float32),
        scratch_shapes=[pltpu.VMEM((G, D), jnp.float32)],
    )(batch3d, h, ln_g, ln_b, W_fc1, b_fc1, bn_g, bn_b, W_fc2, b_fc2)


def kernel(x, edge_index, batch,
           W_msg1_0, b_msg1_0, W_msg2_0, b_msg2_0,
           W_msg1_1, b_msg1_1, W_msg2_1, b_msg2_1,
           ln_g, ln_b, W_fc1, b_fc1, bn_g, bn_b, W_fc2, b_fc2):
    E = edge_index.shape[1]
    EP = ((E + NW * CH - 1) // (NW * CH)) * (NW * CH)

    src = edge_index[0].astype(jnp.int32)
    dst = edge_index[1].astype(jnp.int32)
    pad = EP - E
    src_p = jnp.concatenate([src, jnp.zeros((pad,), jnp.int32)])
    dst_p = jnp.concatenate([dst, jnp.full((pad,), N_TC, jnp.int32)])

    x_p = jnp.pad(x, ((0, N_TC - N), (0, 0)))
    batch_p = jnp.concatenate(
        [batch.astype(jnp.int32), jnp.full((N_TC - N,), jnp.int32(1 << 28), jnp.int32)]
    ).reshape(N_TC // NB, 1, NB)

    sc_aggregate = _make_sc_aggregate(EP)

    b1_0 = b_msg1_0.reshape(1, H)
    b2_0 = b_msg2_0.reshape(1, D)
    b1_1 = b_msg1_1.reshape(1, H)
    b2_1 = b_msg2_1.reshape(1, D)

    agg = sc_aggregate(x_p, src_p, dst_p)
    h1 = _tc_mlp(x_p, agg[0], agg[1], W_msg1_0, b1_0, W_msg2_0, b2_0)
    agg = sc_aggregate(h1, src_p, dst_p)
    h2 = _tc_mlp(h1, agg[0], agg[1], W_msg1_1, b1_1, W_msg2_1, b2_1)

    out = _tc_pool_head(
        batch_p, h2,
        ln_g.reshape(1, D), ln_b.reshape(1, D),
        W_fc1, b_fc1.reshape(1, D),
        bn_g.reshape(1, D), bn_b.reshape(1, D),
        W_fc2, b_fc2.reshape(1, 1),
    )
    return out[:, 0]
